# Initial kernel scaffold; baseline (speedup 1.0000x reference)
import jax
import jax.numpy as jnp
from jax.experimental import pallas as pl


def kernel(x, Q, embeddings):
    raise NotImplementedError("write your pallas kernel here")


# speedup vs baseline: 2.8620x; 2.8620x over previous
import jax
import jax.numpy as jnp
from jax.experimental import pallas as pl
from jax.experimental.pallas import tpu as pltpu

EMBED_DIM = 256
NUM_PTM = 26
NUM_PER = 128
NUM_EMB = NUM_PTM * NUM_PER
TEMP = 0.07
NEG = -1e9

BR = 512


def _quant_kernel(starts_ref, x_ref, emb_ref, q_ref, idx_ref):
    x = x_ref[...]
    emb = emb_ref[...]
    logits = jax.lax.dot_general(
        x, emb, (((1,), (1,)), ((), ())), preferred_element_type=jnp.float32
    )
    starts = starts_ref[...]
    col = jax.lax.broadcasted_iota(jnp.int32, logits.shape, 1)
    mask = (col >= starts) & (col < starts + NUM_PER)
    masked = jnp.where(mask, logits, NEG)
    rowmax = jnp.max(masked, axis=1, keepdims=True)
    e = jnp.where(mask, jnp.exp(masked - rowmax), 0.0)
    s = jnp.sum(e, axis=1, keepdims=True)
    sim = e / s
    q_ref[...] = jax.lax.dot_general(
        sim, emb, (((1,), (0,)), ((), ())), preferred_element_type=jnp.float32
    )
    idx_ref[...] = jnp.min(
        jnp.where(masked == rowmax, col, NUM_EMB), axis=1, keepdims=True
    )


def _unif_kernel(emb_ref, rinv_ref, cinv_ref, out_ref):
    b = pl.program_id(0)
    emb_blk = emb_ref[pl.ds(b * NUM_PER, NUM_PER), :]
    emb = emb_ref[...]
    sim = jax.lax.dot_general(
        emb_blk, emb, (((1,), (1,)), ((), ())), preferred_element_type=jnp.float32
    )
    rinv = rinv_ref[pl.ds(b * NUM_PER, NUM_PER), :]
    sim = sim * rinv * cinv_ref[...]
    row_g = b * NUM_PER + jax.lax.broadcasted_iota(jnp.int32, sim.shape, 0)
    col = jax.lax.broadcasted_iota(jnp.int32, sim.shape, 1)
    sim = jnp.where(col == row_g, NEG, sim)
    e = jnp.exp(sim * (1.0 / TEMP))
    sum_exp = jnp.sum(e, axis=1)
    pos_mask = (col >= b * NUM_PER) & (col < (b + 1) * NUM_PER)
    pos_sum = jnp.sum(jnp.where(pos_mask, e, 0.0), axis=1)
    part = jnp.sum(jnp.log(pos_sum) - jnp.log(sum_exp))

    @pl.when(b == 0)
    def _():
        out_ref[0, 0] = 0.0

    out_ref[0, 0] += part


def kernel(x, Q, embeddings):
    starts = (Q.astype(jnp.int32) * NUM_PER).reshape(-1, 1)
    n_rows = x.shape[0]
    grid = n_rows // BR

    quantized, idx = pl.pallas_call(
        _quant_kernel,
        grid=(grid,),
        in_specs=[
            pl.BlockSpec((BR, 1), lambda i: (i, 0)),
            pl.BlockSpec((BR, EMBED_DIM), lambda i: (i, 0)),
            pl.BlockSpec((NUM_EMB, EMBED_DIM), lambda i: (0, 0)),
        ],
        out_specs=[
            pl.BlockSpec((BR, EMBED_DIM), lambda i: (i, 0)),
            pl.BlockSpec((BR, 1), lambda i: (i, 0)),
        ],
        out_shape=[
            jax.ShapeDtypeStruct((n_rows, EMBED_DIM), jnp.float32),
            jax.ShapeDtypeStruct((n_rows, 1), jnp.int32),
        ],
    )(starts, x, embeddings)

    norms = jnp.sqrt(jnp.sum(embeddings * embeddings, axis=1))
    rinv = (1.0 / norms).reshape(-1, 1)
    cinv = (1.0 / norms).reshape(1, -1)

    total = pl.pallas_call(
        _unif_kernel,
        grid=(NUM_PTM,),
        in_specs=[
            pl.BlockSpec((NUM_EMB, EMBED_DIM), lambda b: (0, 0)),
            pl.BlockSpec((NUM_EMB, 1), lambda b: (0, 0)),
            pl.BlockSpec((1, NUM_EMB), lambda b: (0, 0)),
        ],
        out_specs=pl.BlockSpec(memory_space=pltpu.SMEM),
        out_shape=jax.ShapeDtypeStruct((1, 1), jnp.float32),
    )(embeddings, rinv, cinv)

    uniform_loss = -(total[0, 0] / NUM_EMB)
    loss = jnp.float32(0.0)
    return quantized, loss, uniform_loss, idx.reshape(-1)
